# Initial kernel scaffold; baseline (speedup 1.0000x reference)
import jax
import jax.numpy as jnp
from jax.experimental import pallas as pl


def kernel(x, edge_index, W1, b1, W2, b2, W3, b3):
    raise NotImplementedError("write your pallas kernel here")


# speedup vs baseline: 8.1837x; 8.1837x over previous
import functools

import jax
import jax.numpy as jnp
from jax import lax
from jax.experimental import pallas as pl
from jax.experimental.pallas import tpu as pltpu
from jax.experimental.pallas import tpu_sc as plsc

N = 10000
E = 160000
D = 256
H = 128
NP = 10240
NC = 2
NS = 16
EPS = E // NS
ECH = 128
NFULL = EPS // ECH
EREM = EPS - NFULL * ECH
RPS = NP // NS
RCH = 64
NRC = RPS // RCH
BM = 1024
MB = NP // BM


def _sc_mesh():
    return plsc.VectorSubcoreMesh(
        core_axis_name="c", subcore_axis_name="s",
        num_cores=NC, num_subcores=NS)


def _deg(dst):
    @functools.partial(
        pl.kernel,
        out_type=jax.ShapeDtypeStruct((NP,), jnp.float32),
        mesh=_sc_mesh(),
        scratch_types=[
            pltpu.VMEM((ECH,), jnp.int32),
            pltpu.VMEM((EREM,), jnp.int32),
            pltpu.VMEM((ECH,), jnp.float32),
            pltpu.VMEM((RPS,), jnp.float32),
            pltpu.VMEM_SHARED((NP,), jnp.float32),
        ],
    )
    def k(dst_hbm, dinv_hbm, idx_v, idx_r, ones_v, val_v, deg_sh):
        s = lax.axis_index("s")

        @pl.when(lax.axis_index("c") == 0)
        def _():
            @pl.loop(0, RPS // 16)
            def _(i):
                val_v[pl.ds(i * 16, 16)] = jnp.zeros((16,), jnp.float32)

            pltpu.sync_copy(val_v, deg_sh.at[pl.ds(s * RPS, RPS)])

            @pl.loop(0, ECH // 16)
            def _(i):
                ones_v[pl.ds(i * 16, 16)] = jnp.ones((16,), jnp.float32)

            plsc.subcore_barrier()
            ebase = s * EPS

            @pl.loop(0, NFULL)
            def _(kk):
                pltpu.sync_copy(dst_hbm.at[pl.ds(ebase + kk * ECH, ECH)], idx_v)
                pltpu.sync_copy(ones_v, deg_sh.at[idx_v], add=True)

            pltpu.sync_copy(dst_hbm.at[pl.ds(ebase + NFULL * ECH, EREM)], idx_r)
            pltpu.sync_copy(ones_v.at[pl.ds(0, EREM)], deg_sh.at[idx_r], add=True)
            plsc.subcore_barrier()

            pltpu.sync_copy(deg_sh.at[pl.ds(s * RPS, RPS)],
                            dinv_hbm.at[pl.ds(s * RPS, RPS)])

    return k(dst)


def _prop_body(g_hbm, src_hbm, dst_hbm, out_hbm,
               sidx, didx, sidx_r, didx_r, rows, z_v, acc_sh, sem):
    c = lax.axis_index("c")
    s = lax.axis_index("s")
    coff = c * NP

    @pl.loop(0, RCH)
    def _(i):
        for j in range(H // 16):
            z_v[i, pl.ds(j * 16, 16)] = jnp.zeros((16,), jnp.float32)

    @pl.loop(0, NRC)
    def _(t):
        pltpu.sync_copy(z_v, acc_sh.at[pl.ds(s * RPS + t * RCH, RCH)])

    plsc.subcore_barrier()

    ebase = s * EPS

    @pl.loop(0, NFULL)
    def _(kk):
        b0 = ebase + kk * ECH
        pltpu.sync_copy(src_hbm.at[pl.ds(b0, ECH)], sidx)
        pltpu.sync_copy(dst_hbm.at[pl.ds(b0, ECH)], didx)
        for j in range(ECH // 16):
            sidx[pl.ds(j * 16, 16)] = sidx[pl.ds(j * 16, 16)] + coff
        pltpu.async_copy(g_hbm.at[sidx], rows, sem).wait()
        pltpu.sync_copy(rows, acc_sh.at[didx], add=True)

    b0 = ebase + NFULL * ECH
    pltpu.sync_copy(src_hbm.at[pl.ds(b0, EREM)], sidx_r)
    pltpu.sync_copy(dst_hbm.at[pl.ds(b0, EREM)], didx_r)
    sidx_r[...] = sidx_r[...] + coff
    pltpu.async_copy(g_hbm.at[sidx_r], rows.at[pl.ds(0, EREM)], sem).wait()
    pltpu.sync_copy(rows.at[pl.ds(0, EREM)], acc_sh.at[didx_r], add=True)

    plsc.subcore_barrier()

    @pl.loop(0, NRC)
    def _(t):
        r0 = s * RPS + t * RCH
        pltpu.sync_copy(acc_sh.at[pl.ds(r0, RCH)], out_hbm.at[pl.ds(coff + r0, RCH)])


def _prop(g, src, dst):
    k = functools.partial(
        pl.kernel,
        out_type=jax.ShapeDtypeStruct((NC * NP, H), jnp.float32),
        mesh=_sc_mesh(),
        scratch_types=[
            pltpu.VMEM((ECH,), jnp.int32),
            pltpu.VMEM((ECH,), jnp.int32),
            pltpu.VMEM((EREM,), jnp.int32),
            pltpu.VMEM((EREM,), jnp.int32),
            pltpu.VMEM((ECH, H), jnp.float32),
            pltpu.VMEM((RCH, H), jnp.float32),
            pltpu.VMEM_SHARED((NP, H), jnp.float32),
            pltpu.SemaphoreType.DMA,
        ],
    )(_prop_body)
    return k(g, src, dst)


def _mm1_body(h_ref, w_ref, deg_ref, out_ref):
    kk = pl.program_id(2)
    part = jnp.dot(h_ref[...], w_ref[...], preferred_element_type=jnp.float32)

    @pl.when(kk == 0)
    def _():
        out_ref[...] = part

    @pl.when(kk == 1)
    def _():
        out_ref[...] = (out_ref[...] + part) * lax.rsqrt(deg_ref[...] + 1.0)


def _mm1(h_split, w, dinv2d):
    return pl.pallas_call(
        _mm1_body,
        grid=(MB, NC, NC),
        in_specs=[
            pl.BlockSpec((BM, H), lambda m, n, k: (k * MB + m, 0)),
            pl.BlockSpec((H, H), lambda m, n, k: (k, n)),
            pl.BlockSpec((BM, 1), lambda m, n, k: (m, 0)),
        ],
        out_specs=pl.BlockSpec((BM, H), lambda m, n, k: (n * MB + m, 0)),
        out_shape=jax.ShapeDtypeStruct((NC * NP, H), jnp.float32),
        compiler_params=pltpu.CompilerParams(
            dimension_semantics=("parallel", "parallel", "arbitrary")),
    )(h_split, w, dinv2d)


def _mmf_body(acc_ref, g_ref, b_ref, w_ref, deg_ref, out_ref):
    kk = pl.program_id(2)
    dinv = lax.rsqrt(deg_ref[...] + 1.0)
    h = jnp.maximum(
        dinv * (acc_ref[...] + g_ref[...]) + b_ref[0], 0.0)
    part = jnp.dot(h, w_ref[...], preferred_element_type=jnp.float32)

    @pl.when(kk == 0)
    def _():
        out_ref[...] = part

    @pl.when(kk == 1)
    def _():
        out_ref[...] = (out_ref[...] + part) * dinv


def _mmf(acc, g, b2d, w, dinv2d):
    return pl.pallas_call(
        _mmf_body,
        grid=(MB, NC, NC),
        in_specs=[
            pl.BlockSpec((BM, H), lambda m, n, k: (k * MB + m, 0)),
            pl.BlockSpec((BM, H), lambda m, n, k: (k * MB + m, 0)),
            pl.BlockSpec((1, 1, H), lambda m, n, k: (k, 0, 0)),
            pl.BlockSpec((H, H), lambda m, n, k: (k, n)),
            pl.BlockSpec((BM, 1), lambda m, n, k: (m, 0)),
        ],
        out_specs=pl.BlockSpec((BM, H), lambda m, n, k: (n * MB + m, 0)),
        out_shape=jax.ShapeDtypeStruct((NC * NP, H), jnp.float32),
        compiler_params=pltpu.CompilerParams(
            dimension_semantics=("parallel", "parallel", "arbitrary")),
    )(acc, g, b2d, w, dinv2d)


def _final_body(acc0_ref, g0_ref, acc1_ref, g1_ref, deg_ref, b_ref, out_ref):
    dinv = lax.rsqrt(deg_ref[...] + 1.0)
    p0 = dinv * (acc0_ref[...] + g0_ref[...]) + b_ref[:, :H]
    p1 = dinv * (acc1_ref[...] + g1_ref[...]) + b_ref[:, H:]
    out_ref[...] = jnp.concatenate([p0, p1], axis=1)


def _final(acc, g, dinv2d, b2d):
    return pl.pallas_call(
        _final_body,
        grid=(MB,),
        in_specs=[
            pl.BlockSpec((BM, H), lambda m: (m, 0)),
            pl.BlockSpec((BM, H), lambda m: (m, 0)),
            pl.BlockSpec((BM, H), lambda m: (MB + m, 0)),
            pl.BlockSpec((BM, H), lambda m: (MB + m, 0)),
            pl.BlockSpec((BM, 1), lambda m: (m, 0)),
            pl.BlockSpec((1, D), lambda m: (0, 0)),
        ],
        out_specs=pl.BlockSpec((BM, D), lambda m: (m, 0)),
        out_shape=jax.ShapeDtypeStruct((NP, D), jnp.float32),
        compiler_params=pltpu.CompilerParams(
            dimension_semantics=("parallel",)),
    )(acc, g, acc, g, dinv2d, b2d)


def kernel(x, edge_index, W1, b1, W2, b2, W3, b3):
    src = edge_index[0]
    dst = edge_index[1]
    xp = jnp.pad(x, ((0, NP - N), (0, 0)))
    xs = xp.reshape(NP, NC, H).transpose(1, 0, 2).reshape(NC * NP, H)

    deg = _deg(dst)
    dinv2d = deg.reshape(NP, 1)

    g = _mm1(xs, W1, dinv2d)
    acc = _prop(g, src, dst)
    g = _mmf(acc, g, b1.reshape(NC, 1, H), W2, dinv2d)
    acc = _prop(g, src, dst)
    g = _mmf(acc, g, b2.reshape(NC, 1, H), W3, dinv2d)
    acc = _prop(g, src, dst)

    return _final(acc, g, dinv2d, b3.reshape(1, D))[:N]


# speedup vs baseline: 11.7925x; 1.4410x over previous
import functools

import jax
import jax.numpy as jnp
from jax import lax
from jax.experimental import pallas as pl
from jax.experimental.pallas import tpu as pltpu
from jax.experimental.pallas import tpu_sc as plsc

N = 10000
E = 160000
D = 256
H = 128
NP = 10240
NC = 2
NS = 16
ECH = 128
NCH = E // ECH
CPS = NCH // NS
NXTRA = NCH - CPS * NS
RPS = NP // NS
BM = 1024
MB = NP // BM
CPC = NCH // NC
DCPS = CPC // NS


def _sc_mesh():
    return plsc.VectorSubcoreMesh(
        core_axis_name="c", subcore_axis_name="s",
        num_cores=NC, num_subcores=NS)


def _deg(dst):
    @functools.partial(
        pl.kernel,
        out_type=jax.ShapeDtypeStruct((NC * NP,), jnp.float32),
        mesh=_sc_mesh(),
        scratch_types=[
            pltpu.VMEM((ECH,), jnp.int32),
            pltpu.VMEM((ECH,), jnp.int32),
            pltpu.VMEM((ECH,), jnp.float32),
            pltpu.VMEM((RPS,), jnp.float32),
            pltpu.VMEM_SHARED((NP,), jnp.float32),
            pltpu.SemaphoreType.DMA,
            pltpu.SemaphoreType.DMA,
        ],
    )
    def k(dst_hbm, deg_hbm, didx0, didx1, ones_v, val_v, deg_sh, sem0, sem1):
        c = lax.axis_index("c")
        s = lax.axis_index("s")

        @pl.loop(0, RPS // 16)
        def _(i):
            val_v[pl.ds(i * 16, 16)] = jnp.zeros((16,), jnp.float32)

        pltpu.sync_copy(val_v, deg_sh.at[pl.ds(s * RPS, RPS)])

        @pl.loop(0, ECH // 16)
        def _(i):
            ones_v[pl.ds(i * 16, 16)] = jnp.ones((16,), jnp.float32)

        plsc.subcore_barrier()

        ebase = (c * CPC + s * DCPS) * ECH
        pltpu.sync_copy(dst_hbm.at[pl.ds(ebase, ECH)], didx0)

        @pl.loop(0, DCPS, step=2)
        def _(i):
            pltpu.async_copy(ones_v, deg_sh.at[didx0], sem0, add=True)

            @pl.when(i + 1 < DCPS)
            def _():
                pltpu.sync_copy(
                    dst_hbm.at[pl.ds(ebase + (i + 1) * ECH, ECH)], didx1)

            pltpu.make_async_copy(ones_v, deg_sh.at[didx0], sem0).wait()

            @pl.when(i + 1 < DCPS)
            def _():
                pltpu.async_copy(ones_v, deg_sh.at[didx1], sem1, add=True)

                @pl.when(i + 2 < DCPS)
                def _():
                    pltpu.sync_copy(
                        dst_hbm.at[pl.ds(ebase + (i + 2) * ECH, ECH)], didx0)

                pltpu.make_async_copy(ones_v, deg_sh.at[didx1], sem1).wait()

        @pl.when(s == 0)
        def _():
            pltpu.sync_copy(
                dst_hbm.at[pl.ds((c * CPC + NS * DCPS) * ECH, ECH)], didx0)
            pltpu.sync_copy(ones_v, deg_sh.at[didx0], add=True)

        plsc.subcore_barrier()
        pltpu.sync_copy(deg_sh.at[pl.ds(s * RPS, RPS)],
                        deg_hbm.at[pl.ds(c * NP + s * RPS, RPS)])

    return k(dst)


def _prop_body(g_hbm, src_hbm, dst_hbm, zer_hbm, out_hbm,
               sidx0, sidx1, didx0, didx1, rows0, rows1, acc_sh, sem0, sem1):
    c = lax.axis_index("c")
    s = lax.axis_index("s")
    coff = c * NP

    pltpu.sync_copy(zer_hbm, acc_sh.at[pl.ds(s * RPS, RPS)])
    plsc.subcore_barrier()

    ebase = s * CPS * ECH

    def load_idx(off, sidx, didx):
        pltpu.sync_copy(src_hbm.at[pl.ds(off, ECH)], sidx)
        pltpu.sync_copy(dst_hbm.at[pl.ds(off, ECH)], didx)
        for j in range(ECH // 16):
            sidx[pl.ds(j * 16, 16)] = sidx[pl.ds(j * 16, 16)] + coff

    load_idx(ebase, sidx0, didx0)
    pltpu.async_copy(g_hbm.at[sidx0], rows0, sem0)

    @pl.loop(0, CPS, step=2)
    def _(i):
        load_idx(ebase + (i + 1) * ECH, sidx1, didx1)
        pltpu.async_copy(g_hbm.at[sidx1], rows1, sem1)
        pltpu.make_async_copy(g_hbm.at[sidx0], rows0, sem0).wait()
        pltpu.sync_copy(rows0, acc_sh.at[didx0], add=True)

        @pl.when(i + 2 < CPS)
        def _():
            load_idx(ebase + (i + 2) * ECH, sidx0, didx0)
            pltpu.async_copy(g_hbm.at[sidx0], rows0, sem0)

        pltpu.make_async_copy(g_hbm.at[sidx1], rows1, sem1).wait()
        pltpu.sync_copy(rows1, acc_sh.at[didx1], add=True)

    @pl.when(s < NXTRA)
    def _():
        load_idx((CPS * NS + s) * ECH, sidx0, didx0)
        pltpu.async_copy(g_hbm.at[sidx0], rows0, sem0).wait()
        pltpu.sync_copy(rows0, acc_sh.at[didx0], add=True)

    plsc.subcore_barrier()

    pltpu.sync_copy(acc_sh.at[pl.ds(s * RPS, RPS)],
                    out_hbm.at[pl.ds(coff + s * RPS, RPS)])


def _prop(g, src, dst, zer):
    k = functools.partial(
        pl.kernel,
        out_type=jax.ShapeDtypeStruct((NC * NP, H), jnp.float32),
        mesh=_sc_mesh(),
        scratch_types=[
            pltpu.VMEM((ECH,), jnp.int32),
            pltpu.VMEM((ECH,), jnp.int32),
            pltpu.VMEM((ECH,), jnp.int32),
            pltpu.VMEM((ECH,), jnp.int32),
            pltpu.VMEM((ECH, H), jnp.float32),
            pltpu.VMEM((ECH, H), jnp.float32),
            pltpu.VMEM_SHARED((NP, H), jnp.float32),
            pltpu.SemaphoreType.DMA,
            pltpu.SemaphoreType.DMA,
        ],
    )(_prop_body)
    return k(g, src, dst, zer)


def _mm1_body(h_ref, w_ref, deg0_ref, deg1_ref, out_ref):
    kk = pl.program_id(2)
    part = jnp.dot(h_ref[...], w_ref[...], preferred_element_type=jnp.float32)

    @pl.when(kk == 0)
    def _():
        out_ref[...] = part

    @pl.when(kk == 1)
    def _():
        dinv = lax.rsqrt(deg0_ref[...] + deg1_ref[...] + 1.0)
        out_ref[...] = (out_ref[...] + part) * dinv


def _mm1(h_split, w, deg2d):
    return pl.pallas_call(
        _mm1_body,
        grid=(MB, NC, NC),
        in_specs=[
            pl.BlockSpec((BM, H), lambda m, n, k: (k * MB + m, 0)),
            pl.BlockSpec((H, H), lambda m, n, k: (k, n)),
            pl.BlockSpec((BM, 1), lambda m, n, k: (m, 0)),
            pl.BlockSpec((BM, 1), lambda m, n, k: (MB + m, 0)),
        ],
        out_specs=pl.BlockSpec((BM, H), lambda m, n, k: (n * MB + m, 0)),
        out_shape=jax.ShapeDtypeStruct((NC * NP, H), jnp.float32),
        compiler_params=pltpu.CompilerParams(
            dimension_semantics=("parallel", "parallel", "arbitrary")),
    )(h_split, w, deg2d, deg2d)


def _mmf_body(acc_ref, g_ref, b_ref, w_ref, deg0_ref, deg1_ref, out_ref):
    kk = pl.program_id(2)
    dinv = lax.rsqrt(deg0_ref[...] + deg1_ref[...] + 1.0)
    h = jnp.maximum(
        dinv * (acc_ref[...] + g_ref[...]) + b_ref[0], 0.0)
    part = jnp.dot(h, w_ref[...], preferred_element_type=jnp.float32)

    @pl.when(kk == 0)
    def _():
        out_ref[...] = part

    @pl.when(kk == 1)
    def _():
        out_ref[...] = (out_ref[...] + part) * dinv


def _mmf(acc, g, b3d, w, deg2d):
    return pl.pallas_call(
        _mmf_body,
        grid=(MB, NC, NC),
        in_specs=[
            pl.BlockSpec((BM, H), lambda m, n, k: (k * MB + m, 0)),
            pl.BlockSpec((BM, H), lambda m, n, k: (k * MB + m, 0)),
            pl.BlockSpec((1, 1, H), lambda m, n, k: (k, 0, 0)),
            pl.BlockSpec((H, H), lambda m, n, k: (k, n)),
            pl.BlockSpec((BM, 1), lambda m, n, k: (m, 0)),
            pl.BlockSpec((BM, 1), lambda m, n, k: (MB + m, 0)),
        ],
        out_specs=pl.BlockSpec((BM, H), lambda m, n, k: (n * MB + m, 0)),
        out_shape=jax.ShapeDtypeStruct((NC * NP, H), jnp.float32),
        compiler_params=pltpu.CompilerParams(
            dimension_semantics=("parallel", "parallel", "arbitrary")),
    )(acc, g, b3d, w, deg2d, deg2d)


def _final_body(acc0_ref, g0_ref, acc1_ref, g1_ref, deg0_ref, deg1_ref,
                b_ref, out_ref):
    dinv = lax.rsqrt(deg0_ref[...] + deg1_ref[...] + 1.0)
    p0 = dinv * (acc0_ref[...] + g0_ref[...]) + b_ref[:, :H]
    p1 = dinv * (acc1_ref[...] + g1_ref[...]) + b_ref[:, H:]
    out_ref[...] = jnp.concatenate([p0, p1], axis=1)


def _final(acc, g, deg2d, b2d):
    return pl.pallas_call(
        _final_body,
        grid=(MB,),
        in_specs=[
            pl.BlockSpec((BM, H), lambda m: (m, 0)),
            pl.BlockSpec((BM, H), lambda m: (m, 0)),
            pl.BlockSpec((BM, H), lambda m: (MB + m, 0)),
            pl.BlockSpec((BM, H), lambda m: (MB + m, 0)),
            pl.BlockSpec((BM, 1), lambda m: (m, 0)),
            pl.BlockSpec((BM, 1), lambda m: (MB + m, 0)),
            pl.BlockSpec((1, D), lambda m: (0, 0)),
        ],
        out_specs=pl.BlockSpec((BM, D), lambda m: (m, 0)),
        out_shape=jax.ShapeDtypeStruct((NP, D), jnp.float32),
        compiler_params=pltpu.CompilerParams(
            dimension_semantics=("parallel",)),
    )(acc, g, acc, g, deg2d, deg2d, b2d)


def kernel(x, edge_index, W1, b1, W2, b2, W3, b3):
    src = edge_index[0]
    dst = edge_index[1]
    xp = jnp.pad(x, ((0, NP - N), (0, 0)))
    xs = xp.reshape(NP, NC, H).transpose(1, 0, 2).reshape(NC * NP, H)
    zer = jnp.zeros((RPS, H), jnp.float32)

    deg2d = _deg(dst).reshape(NC * NP, 1)

    g = _mm1(xs, W1, deg2d)
    acc = _prop(g, src, dst, zer)
    g = _mmf(acc, g, b1.reshape(NC, 1, H), W2, deg2d)
    acc = _prop(g, src, dst, zer)
    g = _mmf(acc, g, b2.reshape(NC, 1, H), W3, deg2d)
    acc = _prop(g, src, dst, zer)

    return _final(acc, g, deg2d, b3.reshape(1, D))[:N]


# speedup vs baseline: 14.3605x; 1.2178x over previous
import functools

import jax
import jax.numpy as jnp
from jax import lax
from jax.experimental import pallas as pl
from jax.experimental.pallas import tpu as pltpu
from jax.experimental.pallas import tpu_sc as plsc

N = 10000
E = 160000
D = 256
H = 128
NP = 10240
NC = 2
NS = 16
ECH = 128
NCH = E // ECH
CPS = NCH // NS
NXTRA = NCH - CPS * NS
RPS = NP // NS
BM = 1024
MB = NP // BM
CPC = NCH // NC
DCPS = CPC // NS


def _sc_mesh():
    return plsc.VectorSubcoreMesh(
        core_axis_name="c", subcore_axis_name="s",
        num_cores=NC, num_subcores=NS)


def _deg(dst):
    @functools.partial(
        pl.kernel,
        out_type=jax.ShapeDtypeStruct((NC * NP,), jnp.float32),
        mesh=_sc_mesh(),
        scratch_types=[
            pltpu.VMEM((ECH,), jnp.int32),
            pltpu.VMEM((ECH,), jnp.int32),
            pltpu.VMEM((ECH,), jnp.float32),
            pltpu.VMEM((RPS,), jnp.float32),
            pltpu.VMEM_SHARED((NP,), jnp.float32),
            pltpu.SemaphoreType.DMA,
            pltpu.SemaphoreType.DMA,
        ],
    )
    def k(dst_hbm, deg_hbm, didx0, didx1, ones_v, val_v, deg_sh, sem0, sem1):
        c = lax.axis_index("c")
        s = lax.axis_index("s")

        @pl.loop(0, RPS // 16)
        def _(i):
            val_v[pl.ds(i * 16, 16)] = jnp.zeros((16,), jnp.float32)

        pltpu.sync_copy(val_v, deg_sh.at[pl.ds(s * RPS, RPS)])

        @pl.loop(0, ECH // 16)
        def _(i):
            ones_v[pl.ds(i * 16, 16)] = jnp.ones((16,), jnp.float32)

        plsc.subcore_barrier()

        ebase = (c * CPC + s * DCPS) * ECH
        pltpu.sync_copy(dst_hbm.at[pl.ds(ebase, ECH)], didx0)

        @pl.loop(0, DCPS, step=2)
        def _(i):
            pltpu.async_copy(ones_v, deg_sh.at[didx0], sem0, add=True)

            @pl.when(i + 1 < DCPS)
            def _():
                pltpu.sync_copy(
                    dst_hbm.at[pl.ds(ebase + (i + 1) * ECH, ECH)], didx1)

            pltpu.make_async_copy(ones_v, deg_sh.at[didx0], sem0).wait()

            @pl.when(i + 1 < DCPS)
            def _():
                pltpu.async_copy(ones_v, deg_sh.at[didx1], sem1, add=True)

                @pl.when(i + 2 < DCPS)
                def _():
                    pltpu.sync_copy(
                        dst_hbm.at[pl.ds(ebase + (i + 2) * ECH, ECH)], didx0)

                pltpu.make_async_copy(ones_v, deg_sh.at[didx1], sem1).wait()

        @pl.when(s == 0)
        def _():
            pltpu.sync_copy(
                dst_hbm.at[pl.ds((c * CPC + NS * DCPS) * ECH, ECH)], didx0)
            pltpu.sync_copy(ones_v, deg_sh.at[didx0], add=True)

        plsc.subcore_barrier()
        pltpu.sync_copy(deg_sh.at[pl.ds(s * RPS, RPS)],
                        deg_hbm.at[pl.ds(c * NP + s * RPS, RPS)])

    return k(dst)


def _prop_body(g_hbm, src2_hbm, dst_hbm, zer_hbm, out_hbm,
               sidx_all, didx0, didx1, r0, r1, acc_sh, s0, s1):
    c = lax.axis_index("c")
    s = lax.axis_index("s")

    pltpu.sync_copy(zer_hbm, acc_sh.at[pl.ds(s * RPS, RPS)])
    plsc.subcore_barrier()

    def gslice(j):
        return sidx_all.at[pl.ds(j * ECH, ECH)]

    def ring(cbase, n):
        pltpu.sync_copy(src2_hbm.at[pl.ds(c * E + cbase * ECH, n * ECH)],
                        sidx_all.at[pl.ds(0, n * ECH)])
        pltpu.sync_copy(dst_hbm.at[pl.ds(cbase * ECH, ECH)], didx0)
        pltpu.async_copy(g_hbm.at[gslice(0)], r0, s0)

        @pl.loop(0, n - n % 2, step=2)
        def _(i):
            @pl.when(i + 1 < n)
            def _():
                pltpu.async_copy(g_hbm.at[gslice(i + 1)], r1, s1)
                pltpu.sync_copy(
                    dst_hbm.at[pl.ds((cbase + i + 1) * ECH, ECH)], didx1)

            pltpu.make_async_copy(g_hbm.at[gslice(i)], r0, s0).wait()
            pltpu.sync_copy(r0, acc_sh.at[didx0], add=True)

            @pl.when(i + 2 < n)
            def _():
                pltpu.async_copy(g_hbm.at[gslice(i + 2)], r0, s0)
                pltpu.sync_copy(
                    dst_hbm.at[pl.ds((cbase + i + 2) * ECH, ECH)], didx0)

            pltpu.make_async_copy(g_hbm.at[gslice(i + 1)], r1, s1).wait()
            pltpu.sync_copy(r1, acc_sh.at[didx1], add=True)

        if n % 2:
            pltpu.make_async_copy(g_hbm.at[gslice(n - 1)], r0, s0).wait()
            pltpu.sync_copy(r0, acc_sh.at[didx0], add=True)

    @pl.when(s < NXTRA)
    def _():
        ring(s * (CPS + 1), CPS + 1)

    @pl.when(s >= NXTRA)
    def _():
        ring(NXTRA * (CPS + 1) + (s - NXTRA) * CPS, CPS)

    plsc.subcore_barrier()

    pltpu.sync_copy(acc_sh.at[pl.ds(s * RPS, RPS)],
                    out_hbm.at[pl.ds(c * NP + s * RPS, RPS)])


def _prop(g, src2, dst, zer):
    k = functools.partial(
        pl.kernel,
        out_type=jax.ShapeDtypeStruct((NC * NP, H), jnp.float32),
        mesh=_sc_mesh(),
        scratch_types=[
            pltpu.VMEM(((CPS + 1) * ECH,), jnp.int32),
            pltpu.VMEM((ECH,), jnp.int32),
            pltpu.VMEM((ECH,), jnp.int32),
            pltpu.VMEM((ECH, H), jnp.float32),
            pltpu.VMEM((ECH, H), jnp.float32),
            pltpu.VMEM_SHARED((NP, H), jnp.float32),
            pltpu.SemaphoreType.DMA,
            pltpu.SemaphoreType.DMA,
        ],
    )(_prop_body)
    return k(g, src2, dst, zer)


def _mm1_body(h_ref, w_ref, deg0_ref, deg1_ref, out_ref):
    kk = pl.program_id(2)
    part = jnp.dot(h_ref[...], w_ref[...], preferred_element_type=jnp.float32)

    @pl.when(kk == 0)
    def _():
        out_ref[...] = part

    @pl.when(kk == 1)
    def _():
        dinv = lax.rsqrt(deg0_ref[...] + deg1_ref[...] + 1.0)
        out_ref[...] = (out_ref[...] + part) * dinv


def _mm1(h_split, w, deg2d):
    return pl.pallas_call(
        _mm1_body,
        grid=(MB, NC, NC),
        in_specs=[
            pl.BlockSpec((BM, H), lambda m, n, k: (k * MB + m, 0)),
            pl.BlockSpec((H, H), lambda m, n, k: (k, n)),
            pl.BlockSpec((BM, 1), lambda m, n, k: (m, 0)),
            pl.BlockSpec((BM, 1), lambda m, n, k: (MB + m, 0)),
        ],
        out_specs=pl.BlockSpec((BM, H), lambda m, n, k: (n * MB + m, 0)),
        out_shape=jax.ShapeDtypeStruct((NC * NP, H), jnp.float32),
        compiler_params=pltpu.CompilerParams(
            dimension_semantics=("parallel", "parallel", "arbitrary")),
    )(h_split, w, deg2d, deg2d)


def _mmf_body(acc_ref, g_ref, b_ref, w_ref, deg0_ref, deg1_ref, out_ref):
    kk = pl.program_id(2)
    dinv = lax.rsqrt(deg0_ref[...] + deg1_ref[...] + 1.0)
    h = jnp.maximum(
        dinv * (acc_ref[...] + g_ref[...]) + b_ref[0], 0.0)
    part = jnp.dot(h, w_ref[...], preferred_element_type=jnp.float32)

    @pl.when(kk == 0)
    def _():
        out_ref[...] = part

    @pl.when(kk == 1)
    def _():
        out_ref[...] = (out_ref[...] + part) * dinv


def _mmf(acc, g, b3d, w, deg2d):
    return pl.pallas_call(
        _mmf_body,
        grid=(MB, NC, NC),
        in_specs=[
            pl.BlockSpec((BM, H), lambda m, n, k: (k * MB + m, 0)),
            pl.BlockSpec((BM, H), lambda m, n, k: (k * MB + m, 0)),
            pl.BlockSpec((1, 1, H), lambda m, n, k: (k, 0, 0)),
            pl.BlockSpec((H, H), lambda m, n, k: (k, n)),
            pl.BlockSpec((BM, 1), lambda m, n, k: (m, 0)),
            pl.BlockSpec((BM, 1), lambda m, n, k: (MB + m, 0)),
        ],
        out_specs=pl.BlockSpec((BM, H), lambda m, n, k: (n * MB + m, 0)),
        out_shape=jax.ShapeDtypeStruct((NC * NP, H), jnp.float32),
        compiler_params=pltpu.CompilerParams(
            dimension_semantics=("parallel", "parallel", "arbitrary")),
    )(acc, g, b3d, w, deg2d, deg2d)


def _final_body(acc0_ref, g0_ref, acc1_ref, g1_ref, deg0_ref, deg1_ref,
                b_ref, out_ref):
    dinv = lax.rsqrt(deg0_ref[...] + deg1_ref[...] + 1.0)
    p0 = dinv * (acc0_ref[...] + g0_ref[...]) + b_ref[:, :H]
    p1 = dinv * (acc1_ref[...] + g1_ref[...]) + b_ref[:, H:]
    out_ref[...] = jnp.concatenate([p0, p1], axis=1)


def _final(acc, g, deg2d, b2d):
    return pl.pallas_call(
        _final_body,
        grid=(MB,),
        in_specs=[
            pl.BlockSpec((BM, H), lambda m: (m, 0)),
            pl.BlockSpec((BM, H), lambda m: (m, 0)),
            pl.BlockSpec((BM, H), lambda m: (MB + m, 0)),
            pl.BlockSpec((BM, H), lambda m: (MB + m, 0)),
            pl.BlockSpec((BM, 1), lambda m: (m, 0)),
            pl.BlockSpec((BM, 1), lambda m: (MB + m, 0)),
            pl.BlockSpec((1, D), lambda m: (0, 0)),
        ],
        out_specs=pl.BlockSpec((BM, D), lambda m: (m, 0)),
        out_shape=jax.ShapeDtypeStruct((NP, D), jnp.float32),
        compiler_params=pltpu.CompilerParams(
            dimension_semantics=("parallel",)),
    )(acc, g, acc, g, deg2d, deg2d, b2d)


def kernel(x, edge_index, W1, b1, W2, b2, W3, b3):
    src = edge_index[0]
    dst = edge_index[1]
    xp = jnp.pad(x, ((0, NP - N), (0, 0)))
    xs = xp.reshape(NP, NC, H).transpose(1, 0, 2).reshape(NC * NP, H)
    zer = jnp.zeros((RPS, H), jnp.float32)
    src2 = jnp.concatenate([src, src + NP])

    deg2d = _deg(dst).reshape(NC * NP, 1)

    g = _mm1(xs, W1, deg2d)
    acc = _prop(g, src2, dst, zer)
    g = _mmf(acc, g, b1.reshape(NC, 1, H), W2, deg2d)
    acc = _prop(g, src2, dst, zer)
    g = _mmf(acc, g, b2.reshape(NC, 1, H), W3, deg2d)
    acc = _prop(g, src2, dst, zer)

    return _final(acc, g, deg2d, b3.reshape(1, D))[:N]
